# baseline (speedup 1.0000x reference)
import jax
import jax.numpy as jnp
from jax.experimental import pallas as pl
from jax.experimental.pallas import tpu as pltpu

N_TOK = 32768
D_IN = 768
D_HID = 256
N_EXP = 64
TOP_K = 8
TILE = 512
NUM_TILES = N_TOK // TILE


def _gating_body(x_ref, w1_ref, b1_ref, w2_ref, b2_ref,
                 idx_ref, gate_ref, loss_ref,
                 lt_ref, accp_ref, accc_ref):
    i = pl.program_id(0)

    @pl.when(i == 0)
    def _init():
        accp_ref[...] = jnp.zeros_like(accp_ref)
        accc_ref[...] = jnp.zeros_like(accc_ref)
        loss_ref[...] = jnp.zeros((1, 1), jnp.float32)

    logits_t = lt_ref[...]

    m = jnp.max(logits_t, axis=0, keepdims=True)
    e = jnp.exp(logits_t - m)
    s = jnp.sum(e, axis=0, keepdims=True)
    rs = 1.0 / s
    psum = jax.lax.dot_general(
        e, rs, (((1,), (1,)), ((), ())),
        preferred_element_type=jnp.float32)

    iota_e = jax.lax.broadcasted_iota(jnp.int32, logits_t.shape, 0)
    neg = jnp.float32(-jnp.finfo(jnp.float32).max)
    work = logits_t
    idx_rows = []
    val_rows = []
    for _ in range(TOP_K):
        mk = jnp.max(work, axis=0, keepdims=True)
        key = jnp.where(work == mk, iota_e, N_EXP)
        ik = jnp.min(key, axis=0, keepdims=True)
        work = jnp.where(iota_e == ik, neg, work)
        idx_rows.append(ik)
        val_rows.append(mk)

    idx_ref[...] = jnp.concatenate(idx_rows, axis=0)
    v = jnp.concatenate(val_rows, axis=0)
    g = jnp.exp(v - v[0:1, :])
    gate_ref[...] = g / jnp.sum(g, axis=0, keepdims=True)

    selmask = (work != logits_t).astype(jnp.float32)
    ones_t = jnp.ones((1, TILE), jnp.float32)
    cnt = jax.lax.dot_general(
        selmask, ones_t, (((1,), (1,)), ((), ())),
        preferred_element_type=jnp.float32)

    @pl.when(i > 0)
    def _accumulate():
        accp_ref[...] += psum
        accc_ref[...] += cnt

    x = x_ref[...]
    h = jnp.maximum(
        jnp.dot(x, w1_ref[...], preferred_element_type=jnp.float32)
        + b1_ref[...], 0.0)
    lt_ref[...] = jax.lax.dot_general(
        w2_ref[...], h, (((0,), (1,)), ((), ())),
        preferred_element_type=jnp.float32) + b2_ref[...]

    @pl.when(i == NUM_TILES)
    def _finalize():
        loss_ref[...] = (N_EXP / (N_TOK * N_TOK)) * jnp.sum(
            accp_ref[...] * accc_ref[...], axis=0, keepdims=True)


def kernel(x, W1, b1, W2, b2):
    b1r = b1.reshape(1, D_HID)
    b2r = b2.reshape(N_EXP, 1)
    last = NUM_TILES - 1
    idx_t, gates_t, loss = pl.pallas_call(
        _gating_body,
        grid=(NUM_TILES + 1,),
        in_specs=[
            pl.BlockSpec((TILE, D_IN), lambda i: (jnp.minimum(i, last), 0)),
            pl.BlockSpec((D_IN, D_HID), lambda i: (0, 0)),
            pl.BlockSpec((1, D_HID), lambda i: (0, 0)),
            pl.BlockSpec((D_HID, N_EXP), lambda i: (0, 0)),
            pl.BlockSpec((N_EXP, 1), lambda i: (0, 0)),
        ],
        out_specs=[
            pl.BlockSpec((TOP_K, TILE), lambda i: (0, jnp.maximum(i - 1, 0))),
            pl.BlockSpec((TOP_K, TILE), lambda i: (0, jnp.maximum(i - 1, 0))),
            pl.BlockSpec((1, 1), lambda i: (0, 0)),
        ],
        out_shape=[
            jax.ShapeDtypeStruct((TOP_K, N_TOK), jnp.int32),
            jax.ShapeDtypeStruct((TOP_K, N_TOK), jnp.float32),
            jax.ShapeDtypeStruct((1, 1), jnp.float32),
        ],
        scratch_shapes=[pltpu.VMEM((N_EXP, TILE), jnp.float32),
                        pltpu.VMEM((N_EXP, 1), jnp.float32),
                        pltpu.VMEM((N_EXP, 1), jnp.float32)],
    )(x, W1, b1r, W2, b2r)
    return idx_t.T, gates_t.T, loss.reshape(())


# speedup vs baseline: 1.0013x; 1.0013x over previous
import jax
import jax.numpy as jnp
from jax.experimental import pallas as pl
from jax.experimental.pallas import tpu as pltpu

N_TOK = 32768
D_IN = 768
D_HID = 256
N_EXP = 64
TOP_K = 8
TILE = 512
NUM_TILES = N_TOK // TILE


def _gating_body(x_ref, w1_ref, b1_ref, w2_ref, b2_ref,
                 idx_ref, gate_ref, loss_ref,
                 lt_ref, accp_ref, accc_ref):
    i = pl.program_id(0)

    @pl.when(i == 0)
    def _init():
        accp_ref[...] = jnp.zeros_like(accp_ref)
        accc_ref[...] = jnp.zeros_like(accc_ref)
        loss_ref[...] = jnp.zeros((1, 1), jnp.float32)

    logits_t = lt_ref[...]

    m = jnp.max(logits_t, axis=0, keepdims=True)
    e = jnp.exp(logits_t - m)
    s = jnp.sum(e, axis=0, keepdims=True)
    rs = 1.0 / s
    psum = jax.lax.dot_general(
        e, rs, (((1,), (1,)), ((), ())),
        preferred_element_type=jnp.float32)

    iota_e = jax.lax.broadcasted_iota(jnp.int32, logits_t.shape, 0)
    neg = jnp.float32(-jnp.finfo(jnp.float32).max)
    work = logits_t
    m0 = None
    gsum = None
    for k in range(TOP_K):
        mk = jnp.max(work, axis=0, keepdims=True)
        key = jnp.where(work == mk, iota_e, N_EXP)
        ik = jnp.min(key, axis=0, keepdims=True)
        work = jnp.where(iota_e == ik, neg, work)
        idx_ref[k:k + 1, :] = ik
        if k == 0:
            m0 = mk
            gk = jnp.ones_like(mk)
            gsum = gk
        else:
            gk = jnp.exp(mk - m0)
            gsum = gsum + gk
        gate_ref[k:k + 1, :] = gk
    gate_ref[...] = gate_ref[...] * (1.0 / gsum)

    selmask = (work == neg).astype(jnp.float32)
    ones_t = jnp.ones((1, TILE), jnp.float32)
    cnt = jax.lax.dot_general(
        selmask, ones_t, (((1,), (1,)), ((), ())),
        preferred_element_type=jnp.float32)

    @pl.when(i > 0)
    def _accumulate():
        accp_ref[...] += psum
        accc_ref[...] += cnt

    x = x_ref[...]
    h = jnp.maximum(
        jnp.dot(x, w1_ref[...], preferred_element_type=jnp.float32)
        + b1_ref[...], 0.0)
    lt_ref[...] = jax.lax.dot_general(
        w2_ref[...], h, (((0,), (1,)), ((), ())),
        preferred_element_type=jnp.float32) + b2_ref[...]

    @pl.when(i == NUM_TILES)
    def _finalize():
        loss_ref[...] = (N_EXP / (N_TOK * N_TOK)) * jnp.sum(
            accp_ref[...] * accc_ref[...], axis=0, keepdims=True)


def kernel(x, W1, b1, W2, b2):
    b1r = b1.reshape(1, D_HID)
    b2r = b2.reshape(N_EXP, 1)
    last = NUM_TILES - 1
    idx_t, gates_t, loss = pl.pallas_call(
        _gating_body,
        grid=(NUM_TILES + 1,),
        in_specs=[
            pl.BlockSpec((TILE, D_IN), lambda i: (jnp.minimum(i, last), 0)),
            pl.BlockSpec((D_IN, D_HID), lambda i: (0, 0)),
            pl.BlockSpec((1, D_HID), lambda i: (0, 0)),
            pl.BlockSpec((D_HID, N_EXP), lambda i: (0, 0)),
            pl.BlockSpec((N_EXP, 1), lambda i: (0, 0)),
        ],
        out_specs=[
            pl.BlockSpec((TOP_K, TILE), lambda i: (0, jnp.maximum(i - 1, 0))),
            pl.BlockSpec((TOP_K, TILE), lambda i: (0, jnp.maximum(i - 1, 0))),
            pl.BlockSpec((1, 1), lambda i: (0, 0)),
        ],
        out_shape=[
            jax.ShapeDtypeStruct((TOP_K, N_TOK), jnp.int32),
            jax.ShapeDtypeStruct((TOP_K, N_TOK), jnp.float32),
            jax.ShapeDtypeStruct((1, 1), jnp.float32),
        ],
        scratch_shapes=[pltpu.VMEM((N_EXP, TILE), jnp.float32),
                        pltpu.VMEM((N_EXP, 1), jnp.float32),
                        pltpu.VMEM((N_EXP, 1), jnp.float32)],
    )(x, W1, b1r, W2, b2r)
    return idx_t.T, gates_t.T, loss.reshape(())


# speedup vs baseline: 1.1482x; 1.1467x over previous
import jax
import jax.numpy as jnp
from jax.experimental import pallas as pl
from jax.experimental.pallas import tpu as pltpu

N_TOK = 32768
D_IN = 768
D_HID = 256
N_EXP = 64
TOP_K = 8
TILE = 512
NUM_TILES = N_TOK // TILE


def _gating_body(x_ref, w1_ref, b1_ref, w2_ref, b2_ref,
                 idx_ref, gate_ref, loss_ref,
                 lt_ref, accp_ref, accc_ref):
    i = pl.program_id(0)

    @pl.when(i == 0)
    def _init():
        accp_ref[...] = jnp.zeros_like(accp_ref)
        accc_ref[...] = jnp.zeros_like(accc_ref)
        loss_ref[...] = jnp.zeros((1, 1), jnp.float32)

    logits_t = lt_ref[...]

    m = jnp.max(logits_t, axis=0, keepdims=True)
    e = jnp.exp(logits_t - m)
    s = jnp.sum(e, axis=0, keepdims=True)
    rs = 1.0 / s
    psum = jax.lax.dot_general(
        e, rs, (((1,), (1,)), ((), ())),
        preferred_element_type=jnp.float32)

    iota_e = jax.lax.broadcasted_iota(jnp.int32, logits_t.shape, 0)
    neg = jnp.float32(-jnp.finfo(jnp.float32).max)
    work = logits_t
    m0 = None
    gsum = None
    for k in range(TOP_K):
        mk = jnp.max(work, axis=0, keepdims=True)
        key = jnp.where(work == mk, iota_e, N_EXP)
        ik = jnp.min(key, axis=0, keepdims=True)
        work = jnp.where(iota_e == ik, neg, work)
        idx_ref[k:k + 1, :] = ik
        if k == 0:
            m0 = mk
            gk = jnp.ones_like(mk)
            gsum = gk
        else:
            gk = jnp.exp(mk - m0)
            gsum = gsum + gk
        gate_ref[k:k + 1, :] = gk
    gate_ref[...] = gate_ref[...] * (1.0 / gsum)

    selmask = (work == neg).astype(jnp.float32)
    ones_t = jnp.ones((1, TILE), jnp.float32)
    cnt = jax.lax.dot_general(
        selmask, ones_t, (((1,), (1,)), ((), ())),
        preferred_element_type=jnp.float32)

    x = x_ref[...]
    h = jnp.maximum(
        jnp.dot(x, w1_ref[...], preferred_element_type=jnp.float32)
        + b1_ref[...], 0.0)
    lt_ref[...] = jax.lax.dot_general(
        w2_ref[...], h, (((0,), (1,)), ((), ())),
        preferred_element_type=jnp.float32) + b2_ref[...]

    @pl.when(i > 0)
    def _accumulate():
        accp_ref[...] += psum
        accc_ref[...] += cnt

    @pl.when(i == NUM_TILES)
    def _finalize():
        loss_ref[...] = (N_EXP / (N_TOK * N_TOK)) * jnp.sum(
            accp_ref[...] * accc_ref[...], axis=0, keepdims=True)


def kernel(x, W1, b1, W2, b2):
    b1r = b1.reshape(1, D_HID)
    b2r = b2.reshape(N_EXP, 1)
    last = NUM_TILES - 1
    idx_t, gates_t, loss = pl.pallas_call(
        _gating_body,
        grid=(NUM_TILES + 1,),
        in_specs=[
            pl.BlockSpec((TILE, D_IN), lambda i: (jnp.minimum(i, last), 0)),
            pl.BlockSpec((D_IN, D_HID), lambda i: (0, 0)),
            pl.BlockSpec((1, D_HID), lambda i: (0, 0)),
            pl.BlockSpec((D_HID, N_EXP), lambda i: (0, 0)),
            pl.BlockSpec((N_EXP, 1), lambda i: (0, 0)),
        ],
        out_specs=[
            pl.BlockSpec((TOP_K, TILE), lambda i: (0, jnp.maximum(i - 1, 0))),
            pl.BlockSpec((TOP_K, TILE), lambda i: (0, jnp.maximum(i - 1, 0))),
            pl.BlockSpec((1, 1), lambda i: (0, 0)),
        ],
        out_shape=[
            jax.ShapeDtypeStruct((TOP_K, N_TOK), jnp.int32),
            jax.ShapeDtypeStruct((TOP_K, N_TOK), jnp.float32),
            jax.ShapeDtypeStruct((1, 1), jnp.float32),
        ],
        scratch_shapes=[pltpu.VMEM((N_EXP, TILE), jnp.float32),
                        pltpu.VMEM((N_EXP, 1), jnp.float32),
                        pltpu.VMEM((N_EXP, 1), jnp.float32)],
    )(x, W1, b1r, W2, b2r)
    return idx_t.T, gates_t.T, loss.reshape(())


# speedup vs baseline: 1.4853x; 1.2935x over previous
import jax
import jax.numpy as jnp
from jax.experimental import pallas as pl
from jax.experimental.pallas import tpu as pltpu

N_TOK = 32768
D_IN = 768
D_HID = 256
N_EXP = 64
TOP_K = 8
TILE = 1024
NUM_TILES = N_TOK // TILE


def _gating_body(x_ref, w1_ref, b1_ref, w2_ref, b2_ref,
                 idx_ref, gate_ref, loss_ref,
                 lt_ref, accp_ref, accc_ref):
    i = pl.program_id(0)

    @pl.when(i == 0)
    def _init():
        accp_ref[...] = jnp.zeros_like(accp_ref)
        accc_ref[...] = jnp.zeros_like(accc_ref)
        loss_ref[...] = jnp.zeros((1, 1), jnp.float32)

    logits_t = lt_ref[...]

    m = jnp.max(logits_t, axis=0, keepdims=True)
    e = jnp.exp(logits_t - m)
    s = jnp.sum(e, axis=0, keepdims=True)
    rs = 1.0 / s
    psum = jax.lax.dot_general(
        e, rs, (((1,), (1,)), ((), ())),
        preferred_element_type=jnp.float32)

    iota_e = jax.lax.broadcasted_iota(jnp.int32, logits_t.shape, 0)
    neg = jnp.float32(-jnp.finfo(jnp.float32).max)
    work = logits_t
    m0 = None
    gsum = None
    for k in range(TOP_K):
        mk = jnp.max(work, axis=0, keepdims=True)
        key = jnp.where(work == mk, iota_e, N_EXP)
        ik = jnp.min(key, axis=0, keepdims=True)
        work = jnp.where(iota_e == ik, neg, work)
        idx_ref[k:k + 1, :] = ik
        if k == 0:
            m0 = mk
            gk = jnp.ones_like(mk)
            gsum = gk
        else:
            gk = jnp.exp(mk - m0)
            gsum = gsum + gk
        gate_ref[k:k + 1, :] = gk
    gate_ref[...] = gate_ref[...] * (1.0 / gsum)

    selmask = (work == neg).astype(jnp.float32)
    ones_t = jnp.ones((1, TILE), jnp.float32)
    cnt = jax.lax.dot_general(
        selmask, ones_t, (((1,), (1,)), ((), ())),
        preferred_element_type=jnp.float32)

    x = x_ref[...]
    h = jnp.maximum(
        jnp.dot(x, w1_ref[...], preferred_element_type=jnp.float32)
        + b1_ref[...], 0.0)
    lt_ref[...] = jax.lax.dot_general(
        w2_ref[...], h, (((0,), (1,)), ((), ())),
        preferred_element_type=jnp.float32) + b2_ref[...]

    @pl.when(i > 0)
    def _accumulate():
        accp_ref[...] += psum
        accc_ref[...] += cnt

    @pl.when(i == NUM_TILES)
    def _finalize():
        loss_ref[...] = (N_EXP / (N_TOK * N_TOK)) * jnp.sum(
            accp_ref[...] * accc_ref[...], axis=0, keepdims=True)


def kernel(x, W1, b1, W2, b2):
    b1r = b1.reshape(1, D_HID)
    b2r = b2.reshape(N_EXP, 1)
    last = NUM_TILES - 1
    idx_t, gates_t, loss = pl.pallas_call(
        _gating_body,
        grid=(NUM_TILES + 1,),
        in_specs=[
            pl.BlockSpec((TILE, D_IN), lambda i: (jnp.minimum(i, last), 0)),
            pl.BlockSpec((D_IN, D_HID), lambda i: (0, 0)),
            pl.BlockSpec((1, D_HID), lambda i: (0, 0)),
            pl.BlockSpec((D_HID, N_EXP), lambda i: (0, 0)),
            pl.BlockSpec((N_EXP, 1), lambda i: (0, 0)),
        ],
        out_specs=[
            pl.BlockSpec((TOP_K, TILE), lambda i: (0, jnp.maximum(i - 1, 0))),
            pl.BlockSpec((TOP_K, TILE), lambda i: (0, jnp.maximum(i - 1, 0))),
            pl.BlockSpec((1, 1), lambda i: (0, 0)),
        ],
        out_shape=[
            jax.ShapeDtypeStruct((TOP_K, N_TOK), jnp.int32),
            jax.ShapeDtypeStruct((TOP_K, N_TOK), jnp.float32),
            jax.ShapeDtypeStruct((1, 1), jnp.float32),
        ],
        scratch_shapes=[pltpu.VMEM((N_EXP, TILE), jnp.float32),
                        pltpu.VMEM((N_EXP, 1), jnp.float32),
                        pltpu.VMEM((N_EXP, 1), jnp.float32)],
    )(x, W1, b1r, W2, b2r)
    return idx_t.T, gates_t.T, loss.reshape(())


# speedup vs baseline: 1.4974x; 1.0081x over previous
import jax
import jax.numpy as jnp
from jax.experimental import pallas as pl
from jax.experimental.pallas import tpu as pltpu

N_TOK = 32768
D_IN = 768
D_HID = 256
N_EXP = 64
TOP_K = 8
TILE = 1024
NUM_TILES = N_TOK // TILE


def _gating_body(xa_ref, xb_ref, xc_ref, w1_ref, b1_ref, w2_ref, b2_ref,
                 idx_ref, gate_ref, loss_ref,
                 lt_ref, accp_ref, accc_ref):
    i = pl.program_id(0)

    @pl.when(i == 0)
    def _init():
        accp_ref[...] = jnp.zeros_like(accp_ref)
        accc_ref[...] = jnp.zeros_like(accc_ref)
        loss_ref[...] = jnp.zeros((1, 1), jnp.float32)

    logits_t = lt_ref[...]

    m = jnp.max(logits_t, axis=0, keepdims=True)
    e = jnp.exp(logits_t - m)
    s = jnp.sum(e, axis=0, keepdims=True)
    rs = 1.0 / s
    psum = jax.lax.dot_general(
        e, rs, (((1,), (1,)), ((), ())),
        preferred_element_type=jnp.float32)

    iota_e = jax.lax.broadcasted_iota(jnp.int32, logits_t.shape, 0)
    neg = jnp.float32(-jnp.finfo(jnp.float32).max)
    work = logits_t
    m0 = None
    gsum = None
    for k in range(TOP_K):
        mk = jnp.max(work, axis=0, keepdims=True)
        key = jnp.where(work == mk, iota_e, N_EXP)
        ik = jnp.min(key, axis=0, keepdims=True)
        work = jnp.where(iota_e == ik, neg, work)
        idx_ref[k:k + 1, :] = ik
        if k == 0:
            m0 = mk
            gk = jnp.ones_like(mk)
            gsum = gk
        else:
            gk = jnp.exp(mk - m0)
            gsum = gsum + gk
        gate_ref[k:k + 1, :] = gk
    gate_ref[...] = gate_ref[...] * (1.0 / gsum)

    selmask = (work == neg).astype(jnp.float32)
    ones_t = jnp.ones((1, TILE), jnp.float32)
    cnt = jax.lax.dot_general(
        selmask, ones_t, (((1,), (1,)), ((), ())),
        preferred_element_type=jnp.float32)

    acc = jnp.dot(xa_ref[...], w1_ref[0:D_IN // 3, :],
                  preferred_element_type=jnp.float32)
    acc += jnp.dot(xb_ref[...], w1_ref[D_IN // 3:2 * D_IN // 3, :],
                   preferred_element_type=jnp.float32)
    acc += jnp.dot(xc_ref[...], w1_ref[2 * D_IN // 3:, :],
                   preferred_element_type=jnp.float32)
    h = jnp.maximum(acc + b1_ref[...], 0.0)
    lt_ref[...] = jax.lax.dot_general(
        w2_ref[...], h, (((0,), (1,)), ((), ())),
        preferred_element_type=jnp.float32) + b2_ref[...]

    @pl.when(i > 0)
    def _accumulate():
        accp_ref[...] += psum
        accc_ref[...] += cnt

    @pl.when(i == NUM_TILES)
    def _finalize():
        loss_ref[...] = (N_EXP / (N_TOK * N_TOK)) * jnp.sum(
            accp_ref[...] * accc_ref[...], axis=0, keepdims=True)


def kernel(x, W1, b1, W2, b2):
    b1r = b1.reshape(1, D_HID)
    b2r = b2.reshape(N_EXP, 1)
    last = NUM_TILES - 1
    idx_t, gates_t, loss = pl.pallas_call(
        _gating_body,
        grid=(NUM_TILES + 1,),
        in_specs=[
            pl.BlockSpec((TILE, D_IN // 3),
                         lambda i: (jnp.minimum(i, last), 0)),
            pl.BlockSpec((TILE, D_IN // 3),
                         lambda i: (jnp.minimum(i, last), 1)),
            pl.BlockSpec((TILE, D_IN // 3),
                         lambda i: (jnp.minimum(i, last), 2)),
            pl.BlockSpec((D_IN, D_HID), lambda i: (0, 0)),
            pl.BlockSpec((1, D_HID), lambda i: (0, 0)),
            pl.BlockSpec((D_HID, N_EXP), lambda i: (0, 0)),
            pl.BlockSpec((N_EXP, 1), lambda i: (0, 0)),
        ],
        out_specs=[
            pl.BlockSpec((TOP_K, TILE), lambda i: (0, jnp.maximum(i - 1, 0))),
            pl.BlockSpec((TOP_K, TILE), lambda i: (0, jnp.maximum(i - 1, 0))),
            pl.BlockSpec((1, 1), lambda i: (0, 0)),
        ],
        out_shape=[
            jax.ShapeDtypeStruct((TOP_K, N_TOK), jnp.int32),
            jax.ShapeDtypeStruct((TOP_K, N_TOK), jnp.float32),
            jax.ShapeDtypeStruct((1, 1), jnp.float32),
        ],
        scratch_shapes=[pltpu.VMEM((N_EXP, TILE), jnp.float32),
                        pltpu.VMEM((N_EXP, 1), jnp.float32),
                        pltpu.VMEM((N_EXP, 1), jnp.float32)],
    )(x, x, x, W1, b1r, W2, b2r)
    return idx_t.T, gates_t.T, loss.reshape(())


# speedup vs baseline: 1.5651x; 1.0452x over previous
import jax
import jax.numpy as jnp
from jax.experimental import pallas as pl
from jax.experimental.pallas import tpu as pltpu

N_TOK = 32768
D_IN = 768
D_HID = 256
N_EXP = 64
TOP_K = 8
TILE = 2048
NUM_TILES = N_TOK // TILE


def _gating_body(xa_ref, xb_ref, xc_ref, w1_ref, b1_ref, w2_ref, b2_ref,
                 idx_ref, gate_ref, loss_ref,
                 lt_ref, accp_ref, accc_ref):
    i = pl.program_id(0)

    @pl.when(i == 0)
    def _init():
        accp_ref[...] = jnp.zeros_like(accp_ref)
        accc_ref[...] = jnp.zeros_like(accc_ref)
        loss_ref[...] = jnp.zeros((1, 1), jnp.float32)

    logits_t = lt_ref[...]

    m = jnp.max(logits_t, axis=0, keepdims=True)
    e = jnp.exp(logits_t - m)
    s = jnp.sum(e, axis=0, keepdims=True)
    rs = 1.0 / s
    psum = jax.lax.dot_general(
        e, rs, (((1,), (1,)), ((), ())),
        preferred_element_type=jnp.float32)

    iota_e = jax.lax.broadcasted_iota(jnp.int32, logits_t.shape, 0)
    neg = jnp.float32(-jnp.finfo(jnp.float32).max)
    work = logits_t
    m0 = None
    gsum = None
    for k in range(TOP_K):
        mk = jnp.max(work, axis=0, keepdims=True)
        key = jnp.where(work == mk, iota_e, N_EXP)
        ik = jnp.min(key, axis=0, keepdims=True)
        work = jnp.where(iota_e == ik, neg, work)
        idx_ref[k:k + 1, :] = ik
        if k == 0:
            m0 = mk
            gk = jnp.ones_like(mk)
            gsum = gk
        else:
            gk = jnp.exp(mk - m0)
            gsum = gsum + gk
        gate_ref[k:k + 1, :] = gk
    gate_ref[...] = gate_ref[...] * (1.0 / gsum)

    selmask = (work == neg).astype(jnp.float32)
    ones_t = jnp.ones((1, TILE), jnp.float32)
    cnt = jax.lax.dot_general(
        selmask, ones_t, (((1,), (1,)), ((), ())),
        preferred_element_type=jnp.float32)

    acc = jnp.dot(xa_ref[...], w1_ref[0:D_IN // 3, :],
                  preferred_element_type=jnp.float32)
    acc += jnp.dot(xb_ref[...], w1_ref[D_IN // 3:2 * D_IN // 3, :],
                   preferred_element_type=jnp.float32)
    acc += jnp.dot(xc_ref[...], w1_ref[2 * D_IN // 3:, :],
                   preferred_element_type=jnp.float32)
    h = jnp.maximum(acc + b1_ref[...], 0.0)
    lt_ref[...] = jax.lax.dot_general(
        w2_ref[...], h, (((0,), (1,)), ((), ())),
        preferred_element_type=jnp.float32) + b2_ref[...]

    @pl.when(i > 0)
    def _accumulate():
        accp_ref[...] += psum
        accc_ref[...] += cnt

    @pl.when(i == NUM_TILES)
    def _finalize():
        loss_ref[...] = (N_EXP / (N_TOK * N_TOK)) * jnp.sum(
            accp_ref[...] * accc_ref[...], axis=0, keepdims=True)


def kernel(x, W1, b1, W2, b2):
    b1r = b1.reshape(1, D_HID)
    b2r = b2.reshape(N_EXP, 1)
    last = NUM_TILES - 1
    idx_t, gates_t, loss = pl.pallas_call(
        _gating_body,
        grid=(NUM_TILES + 1,),
        in_specs=[
            pl.BlockSpec((TILE, D_IN // 3),
                         lambda i: (jnp.minimum(i, last), 0)),
            pl.BlockSpec((TILE, D_IN // 3),
                         lambda i: (jnp.minimum(i, last), 1)),
            pl.BlockSpec((TILE, D_IN // 3),
                         lambda i: (jnp.minimum(i, last), 2)),
            pl.BlockSpec((D_IN, D_HID), lambda i: (0, 0)),
            pl.BlockSpec((1, D_HID), lambda i: (0, 0)),
            pl.BlockSpec((D_HID, N_EXP), lambda i: (0, 0)),
            pl.BlockSpec((N_EXP, 1), lambda i: (0, 0)),
        ],
        out_specs=[
            pl.BlockSpec((TOP_K, TILE), lambda i: (0, jnp.maximum(i - 1, 0))),
            pl.BlockSpec((TOP_K, TILE), lambda i: (0, jnp.maximum(i - 1, 0))),
            pl.BlockSpec((1, 1), lambda i: (0, 0)),
        ],
        out_shape=[
            jax.ShapeDtypeStruct((TOP_K, N_TOK), jnp.int32),
            jax.ShapeDtypeStruct((TOP_K, N_TOK), jnp.float32),
            jax.ShapeDtypeStruct((1, 1), jnp.float32),
        ],
        scratch_shapes=[pltpu.VMEM((N_EXP, TILE), jnp.float32),
                        pltpu.VMEM((N_EXP, 1), jnp.float32),
                        pltpu.VMEM((N_EXP, 1), jnp.float32)],
    )(x, x, x, W1, b1r, W2, b2r)
    return idx_t.T, gates_t.T, loss.reshape(())
